# baseline (speedup 1.0000x reference)
import functools

import jax
import jax.numpy as jnp
from jax import lax
from jax.experimental import pallas as pl
from jax.experimental.pallas import tpu as pltpu
from jax.experimental.pallas import tpu_sc as plsc

B = 4
S = 2048
D = 2048
N_ROUTES = 16
NVT = 100

S_BLK = 128
NS_BLK = S // S_BLK

NC = 2
NSUB = 16
NW = NC * NSUB
CHUNK = 8
CPB = 13
NCHUNK = B * CPB
NCHUNK_PAD = 2 * NW
NSLOT = 2 * CHUNK
LANES = 16


def _route_body(x_ref, m_ref, wr_ref, g_ref, acc_ref, cacc_ref):
    s = pl.program_id(0)
    xb = x_ref[...]
    mb = m_ref[...].astype(jnp.float32)
    part = jnp.sum(xb * mb[:, :, None], axis=1)
    cb = jnp.sum(mb, axis=1, keepdims=True)

    @pl.when(s == 0)
    def _():
        acc_ref[...] = part
        cacc_ref[...] = jnp.broadcast_to(cb, (B, 128))

    @pl.when(s > 0)
    def _():
        acc_ref[...] = acc_ref[...] + part
        cacc_ref[...] = cacc_ref[...] + cb

    @pl.when(s == NS_BLK - 1)
    def _finalize():
        cnt = jnp.maximum(cacc_ref[:, 0:1], 1.0)
        sent_all = acc_ref[...] / cnt
        logits = lax.dot_general(
            sent_all, wr_ref[...], (((1,), (1,)), ((), ())),
            precision=lax.Precision.HIGHEST,
            preferred_element_type=jnp.float32)
        z = logits - jnp.max(logits, axis=1, keepdims=True)
        ez = jnp.exp(z)
        p = ez / jnp.sum(ez, axis=1, keepdims=True)

        iota = lax.broadcasted_iota(jnp.int32, (B, N_ROUTES), 1)
        m1 = jnp.max(p, axis=1, keepdims=True)
        i1 = jnp.min(jnp.where(p == m1, iota, N_ROUTES), axis=1, keepdims=True)
        p2 = jnp.where(iota == i1, -1.0, p)
        m2 = jnp.max(p2, axis=1, keepdims=True)
        i2 = jnp.min(jnp.where(p2 == m2, iota, N_ROUTES), axis=1, keepdims=True)

        cq = lax.broadcasted_iota(jnp.int32, (NCHUNK_PAD, NSLOT), 0)
        tq = lax.broadcasted_iota(jnp.int32, (NCHUNK_PAD, NSLOT), 1)
        bq = jnp.minimum(cq // CPB, B - 1)
        kq = cq % CPB
        jg = jnp.minimum(kq * CHUNK + tq // 2, NVT - 1)
        route0g = (tq % 2) == 0
        route0w = tq < CHUNK
        valid = (cq < NCHUNK) & (kq * CHUNK + (tq % CHUNK) < NVT)
        gsel = jnp.zeros((NCHUNK_PAD, NSLOT), jnp.int32)
        wsel = jnp.zeros((NCHUNK_PAD, NSLOT), jnp.float32)
        for bb in range(B):
            t1 = lax.slice(i1, (bb, 0), (bb + 1, 1))
            t2 = lax.slice(i2, (bb, 0), (bb + 1, 1))
            v1 = lax.slice(m1, (bb, 0), (bb + 1, 1))
            v2 = lax.slice(m2, (bb, 0), (bb + 1, 1))
            gsel = gsel + jnp.where(bq == bb, jnp.where(route0g, t1, t2), 0)
            wsel = wsel + jnp.where(bq == bb, jnp.where(route0w, v1, v2), 0.0)
        g_ref[0:NCHUNK_PAD, :] = gsel * NVT + jg
        g_ref[NCHUNK_PAD:2 * NCHUNK_PAD, :] = (
            jnp.where(valid, wsel, 0.0) * float(1 << 24)).astype(jnp.int32)


_route = pl.pallas_call(
    _route_body,
    grid=(NS_BLK,),
    in_specs=[
        pl.BlockSpec((B, S_BLK, D), lambda s: (0, s, 0)),
        pl.BlockSpec((B, S_BLK), lambda s: (0, s)),
        pl.BlockSpec((N_ROUTES, D), lambda s: (0, 0)),
    ],
    out_specs=[
        pl.BlockSpec((2 * NCHUNK_PAD, NSLOT), lambda s: (0, 0)),
    ],
    out_shape=[
        jax.ShapeDtypeStruct((2 * NCHUNK_PAD, NSLOT), jnp.int32),
    ],
    scratch_shapes=[
        pltpu.VMEM((B, D), jnp.float32),
        pltpu.VMEM((B, 128), jnp.float32),
    ],
)

_SPLAT_DNUMS = lax.GatherDimensionNumbers(
    offset_dims=(), collapsed_slice_dims=(0,), start_index_map=(0,))


def _splat(vec, i):
    iv = jnp.full((LANES, 1), i, jnp.int32)
    return lax.gather(vec, iv, _SPLAT_DNUMS, (1,),
                      mode=lax.GatherScatterMode.PROMISE_IN_BOUNDS)


def _combine_chunk(c, t_v, rows, out_v):
    wrow = (t_v[NCHUNK_PAD + c, pl.ds(0, LANES)].astype(jnp.float32)
            * (1.0 / float(1 << 24)))

    def _row(l, carry):
        w0 = _splat(wrow, l)
        w1 = _splat(wrow, CHUNK + l)

        @plsc.parallel_loop(0, D, step=LANES, unroll=16)
        def _col(d):
            a = rows[2 * l, pl.ds(d, LANES)]
            b2 = rows[2 * l + 1, pl.ds(d, LANES)]
            out_v[l, pl.ds(d, LANES)] = a * w0 + b2 * w1

        return carry

    lax.fori_loop(0, CHUNK, _row, 0)


def _store_chunk(c, out_v, out_ref, sem_o):
    bc = c // CPB
    kc = c % CPB
    off = pl.multiple_of(kc * CHUNK, CHUNK)

    @pl.when(kc < CPB - 1)
    def _():
        pltpu.async_copy(out_v.at[pl.ds(0, CHUNK)],
                         out_ref.at[bc, pl.ds(off, CHUNK)], sem_o)

    @pl.when(kc == CPB - 1)
    def _():
        pltpu.async_copy(out_v.at[pl.ds(0, 4)],
                         out_ref.at[bc, pl.ds(off, 4)], sem_o)


def _wait_chunk(c, out_v, out_ref, sem_o):
    bc = c // CPB
    kc = c % CPB
    off = pl.multiple_of(kc * CHUNK, CHUNK)

    @pl.when(kc < CPB - 1)
    def _():
        pltpu.make_async_copy(out_v.at[pl.ds(0, CHUNK)],
                              out_ref.at[bc, pl.ds(off, CHUNK)], sem_o).wait()

    @pl.when(kc == CPB - 1)
    def _():
        pltpu.make_async_copy(out_v.at[pl.ds(0, 4)],
                              out_ref.at[bc, pl.ds(off, 4)], sem_o).wait()


def _combine_body(emb_ref, g_ref, out_ref, t_v,
                  rows_a, rows_b, out_va, out_vb, sem_a, sem_b, sem_o):
    wid = lax.axis_index("s") * NC + lax.axis_index("c")
    c0 = wid
    c1 = wid + NW
    pltpu.sync_copy(g_ref, t_v)
    cp_a = pltpu.async_copy(emb_ref.at[t_v.at[c0]], rows_a, sem_a)

    @pl.when(c1 < NCHUNK)
    def _():
        pltpu.async_copy(emb_ref.at[t_v.at[c1]], rows_b, sem_b)

    cp_a.wait()
    _combine_chunk(c0, t_v, rows_a, out_va)
    _store_chunk(c0, out_va, out_ref, sem_o)

    @pl.when(c1 < NCHUNK)
    def _():
        pltpu.make_async_copy(emb_ref.at[t_v.at[c1]], rows_b, sem_b).wait()
        _combine_chunk(c1, t_v, rows_b, out_vb)
        _store_chunk(c1, out_vb, out_ref, sem_o)
        _wait_chunk(c1, out_vb, out_ref, sem_o)

    _wait_chunk(c0, out_va, out_ref, sem_o)


@functools.cache
def _get_combine():
    return pl.kernel(
        _combine_body,
        out_type=jax.ShapeDtypeStruct((B, NVT, D), jnp.float32),
        mesh=plsc.VectorSubcoreMesh(core_axis_name="c", subcore_axis_name="s",
                                    num_cores=NC, num_subcores=NSUB),
        scratch_types=[
            pltpu.VMEM((2 * NCHUNK_PAD, NSLOT), jnp.int32),
            pltpu.VMEM((NSLOT, D), jnp.float32),
            pltpu.VMEM((NSLOT, D), jnp.float32),
            pltpu.VMEM((CHUNK, D), jnp.float32),
            pltpu.VMEM((CHUNK, D), jnp.float32),
            pltpu.SemaphoreType.DMA,
            pltpu.SemaphoreType.DMA,
            pltpu.SemaphoreType.DMA,
        ],
    )


def kernel(indices, input_ids, inputs_embeds, attention_mask, embedding, W_router):
    (tab,) = _route(inputs_embeds, attention_mask, W_router)
    return _get_combine()(embedding, tab)


# speedup vs baseline: 1.0578x; 1.0578x over previous
import functools

import jax
import jax.numpy as jnp
from jax import lax
from jax.experimental import pallas as pl
from jax.experimental.pallas import tpu as pltpu
from jax.experimental.pallas import tpu_sc as plsc

B = 4
S = 2048
D = 2048
N_ROUTES = 16
NVT = 100

S_BLK = 256
NS_BLK = S // S_BLK

NC = 2
NSUB = 16
NW = NC * NSUB
CHUNK = 8
CPB = 13
NCHUNK = B * CPB
NCHUNK_PAD = 2 * NW
NSLOT = 2 * CHUNK
LANES = 16


def _route_body(x_ref, m_ref, wr_ref, g_ref, acc_ref, cacc_ref):
    s = pl.program_id(0)
    xb = x_ref[...]
    mb = m_ref[...].astype(jnp.float32)
    part = jnp.sum(xb * mb[:, :, None], axis=1)
    cb = jnp.sum(mb, axis=1, keepdims=True)

    @pl.when(s == 0)
    def _():
        acc_ref[...] = part
        cacc_ref[...] = jnp.broadcast_to(cb, (B, 128))

    @pl.when(s > 0)
    def _():
        acc_ref[...] = acc_ref[...] + part
        cacc_ref[...] = cacc_ref[...] + cb

    @pl.when(s == NS_BLK - 1)
    def _finalize():
        cnt = jnp.maximum(cacc_ref[:, 0:1], 1.0)
        sent_all = acc_ref[...] / cnt
        logits = lax.dot_general(
            sent_all, wr_ref[...], (((1,), (1,)), ((), ())),
            precision=lax.Precision.HIGHEST,
            preferred_element_type=jnp.float32)
        z = logits - jnp.max(logits, axis=1, keepdims=True)
        ez = jnp.exp(z)
        p = ez / jnp.sum(ez, axis=1, keepdims=True)

        iota = lax.broadcasted_iota(jnp.int32, (B, N_ROUTES), 1)
        m1 = jnp.max(p, axis=1, keepdims=True)
        i1 = jnp.min(jnp.where(p == m1, iota, N_ROUTES), axis=1, keepdims=True)
        p2 = jnp.where(iota == i1, -1.0, p)
        m2 = jnp.max(p2, axis=1, keepdims=True)
        i2 = jnp.min(jnp.where(p2 == m2, iota, N_ROUTES), axis=1, keepdims=True)

        cq = lax.broadcasted_iota(jnp.int32, (NCHUNK_PAD, NSLOT), 0)
        tq = lax.broadcasted_iota(jnp.int32, (NCHUNK_PAD, NSLOT), 1)
        bq = jnp.minimum(cq // CPB, B - 1)
        kq = cq % CPB
        jg = jnp.minimum(kq * CHUNK + tq // 2, NVT - 1)
        route0g = (tq % 2) == 0
        route0w = tq < CHUNK
        valid = (cq < NCHUNK) & (kq * CHUNK + (tq % CHUNK) < NVT)
        gsel = jnp.zeros((NCHUNK_PAD, NSLOT), jnp.int32)
        wsel = jnp.zeros((NCHUNK_PAD, NSLOT), jnp.float32)
        for bb in range(B):
            t1 = lax.slice(i1, (bb, 0), (bb + 1, 1))
            t2 = lax.slice(i2, (bb, 0), (bb + 1, 1))
            v1 = lax.slice(m1, (bb, 0), (bb + 1, 1))
            v2 = lax.slice(m2, (bb, 0), (bb + 1, 1))
            gsel = gsel + jnp.where(bq == bb, jnp.where(route0g, t1, t2), 0)
            wsel = wsel + jnp.where(bq == bb, jnp.where(route0w, v1, v2), 0.0)
        g_ref[0:NCHUNK_PAD, :] = gsel * NVT + jg
        g_ref[NCHUNK_PAD:2 * NCHUNK_PAD, :] = (
            jnp.where(valid, wsel, 0.0) * float(1 << 24)).astype(jnp.int32)


_route = pl.pallas_call(
    _route_body,
    grid=(NS_BLK,),
    in_specs=[
        pl.BlockSpec((B, S_BLK, D), lambda s: (0, s, 0)),
        pl.BlockSpec((B, S_BLK), lambda s: (0, s)),
        pl.BlockSpec((N_ROUTES, D), lambda s: (0, 0)),
    ],
    out_specs=[
        pl.BlockSpec((2 * NCHUNK_PAD, NSLOT), lambda s: (0, 0)),
    ],
    out_shape=[
        jax.ShapeDtypeStruct((2 * NCHUNK_PAD, NSLOT), jnp.int32),
    ],
    scratch_shapes=[
        pltpu.VMEM((B, D), jnp.float32),
        pltpu.VMEM((B, 128), jnp.float32),
    ],
)

_SPLAT_DNUMS = lax.GatherDimensionNumbers(
    offset_dims=(), collapsed_slice_dims=(0,), start_index_map=(0,))


def _splat(vec, i):
    iv = jnp.full((LANES, 1), i, jnp.int32)
    return lax.gather(vec, iv, _SPLAT_DNUMS, (1,),
                      mode=lax.GatherScatterMode.PROMISE_IN_BOUNDS)


def _combine_chunk(c, t_v, rows, out_v):
    wrow = (t_v[NCHUNK_PAD + c, pl.ds(0, LANES)].astype(jnp.float32)
            * (1.0 / float(1 << 24)))

    def _row(l, carry):
        w0 = _splat(wrow, l)
        w1 = _splat(wrow, CHUNK + l)

        @plsc.parallel_loop(0, D, step=LANES, unroll=16)
        def _col(d):
            a = rows[2 * l, pl.ds(d, LANES)]
            b2 = rows[2 * l + 1, pl.ds(d, LANES)]
            out_v[l, pl.ds(d, LANES)] = a * w0 + b2 * w1

        return carry

    lax.fori_loop(0, CHUNK, _row, 0)


def _store_chunk(c, out_v, out_ref, sem_o):
    bc = c // CPB
    kc = c % CPB
    off = pl.multiple_of(kc * CHUNK, CHUNK)

    @pl.when(kc < CPB - 1)
    def _():
        pltpu.async_copy(out_v.at[pl.ds(0, CHUNK)],
                         out_ref.at[bc, pl.ds(off, CHUNK)], sem_o)

    @pl.when(kc == CPB - 1)
    def _():
        pltpu.async_copy(out_v.at[pl.ds(0, 4)],
                         out_ref.at[bc, pl.ds(off, 4)], sem_o)


def _wait_chunk(c, out_v, out_ref, sem_o):
    bc = c // CPB
    kc = c % CPB
    off = pl.multiple_of(kc * CHUNK, CHUNK)

    @pl.when(kc < CPB - 1)
    def _():
        pltpu.make_async_copy(out_v.at[pl.ds(0, CHUNK)],
                              out_ref.at[bc, pl.ds(off, CHUNK)], sem_o).wait()

    @pl.when(kc == CPB - 1)
    def _():
        pltpu.make_async_copy(out_v.at[pl.ds(0, 4)],
                              out_ref.at[bc, pl.ds(off, 4)], sem_o).wait()


def _combine_body(emb_ref, g_ref, out_ref, t_v,
                  rows_a, rows_b, out_va, out_vb, sem_a, sem_b, sem_o):
    wid = lax.axis_index("s") * NC + lax.axis_index("c")
    c0 = wid
    c1 = wid + NW
    pltpu.sync_copy(g_ref, t_v)
    cp_a = pltpu.async_copy(emb_ref.at[t_v.at[c0]], rows_a, sem_a)

    @pl.when(c1 < NCHUNK)
    def _():
        pltpu.async_copy(emb_ref.at[t_v.at[c1]], rows_b, sem_b)

    cp_a.wait()
    _combine_chunk(c0, t_v, rows_a, out_va)
    _store_chunk(c0, out_va, out_ref, sem_o)

    @pl.when(c1 < NCHUNK)
    def _():
        pltpu.make_async_copy(emb_ref.at[t_v.at[c1]], rows_b, sem_b).wait()
        _combine_chunk(c1, t_v, rows_b, out_vb)
        _store_chunk(c1, out_vb, out_ref, sem_o)
        _wait_chunk(c1, out_vb, out_ref, sem_o)

    _wait_chunk(c0, out_va, out_ref, sem_o)


@functools.cache
def _get_combine():
    return pl.kernel(
        _combine_body,
        out_type=jax.ShapeDtypeStruct((B, NVT, D), jnp.float32),
        mesh=plsc.VectorSubcoreMesh(core_axis_name="c", subcore_axis_name="s",
                                    num_cores=NC, num_subcores=NSUB),
        scratch_types=[
            pltpu.VMEM((2 * NCHUNK_PAD, NSLOT), jnp.int32),
            pltpu.VMEM((NSLOT, D), jnp.float32),
            pltpu.VMEM((NSLOT, D), jnp.float32),
            pltpu.VMEM((CHUNK, D), jnp.float32),
            pltpu.VMEM((CHUNK, D), jnp.float32),
            pltpu.SemaphoreType.DMA,
            pltpu.SemaphoreType.DMA,
            pltpu.SemaphoreType.DMA,
        ],
    )


def kernel(indices, input_ids, inputs_embeds, attention_mask, embedding, W_router):
    (tab,) = _route(inputs_embeds, attention_mask, W_router)
    return _get_combine()(embedding, tab)
